# Initial kernel scaffold; baseline (speedup 1.0000x reference)
import jax
import jax.numpy as jnp
from jax.experimental import pallas as pl


def kernel(x, conv_w, conv_b):
    raise NotImplementedError("write your pallas kernel here")


# speedup vs baseline: 208.4318x; 208.4318x over previous
import functools

import jax
import jax.numpy as jnp
from jax import lax
from jax.experimental import pallas as pl
from jax.experimental.pallas import tpu as pltpu
from jax.experimental.pallas import tpu_sc as plsc

_K = 9
_NBR = _K - 1
_TQ = 256
_TR = 512


def _topk_body(xt_ref, xf_ref, idx_ref):
    b = pl.program_id(0)
    qi = pl.program_id(1)
    xt = xt_ref[0]
    xf = xf_ref[0]
    n = xf.shape[1]
    dot = lax.dot_general(xt, xf, (((1,), (0,)), ((), ())),
                          preferred_element_type=jnp.float32)
    nk = jnp.sum(xf * xf, axis=0, keepdims=True)
    r = nk - 2.0 * dot
    lane = lax.broadcasted_iota(jnp.int32, r.shape, 1)
    row = qi * _TQ + lax.broadcasted_iota(jnp.int32, r.shape, 0)
    inf = jnp.float32(jnp.inf)
    r = jnp.where(lane == row, inf, r)
    lane8 = lax.broadcasted_iota(jnp.int32, (r.shape[0], _NBR), 1)
    acc = jnp.zeros((r.shape[0], _NBR), jnp.int32)
    base = b * n
    for k in range(_NBR):
        m = jnp.min(r, axis=1, keepdims=True)
        isel = jnp.min(jnp.where(r == m, lane, n), axis=1,
                       keepdims=True)
        acc = jnp.where(lane8 == k, isel + base, acc)
        r = jnp.where(lane == isel, inf, r)
    idx_ref[0] = acc


def _topk_call(xt, xf):
    b, n, c = xt.shape
    return pl.pallas_call(
        _topk_body,
        grid=(b, n // _TQ),
        in_specs=[
            pl.BlockSpec((1, _TQ, c), lambda i, q: (i, q, 0)),
            pl.BlockSpec((1, c, n), lambda i, q: (i, 0, 0)),
        ],
        out_specs=pl.BlockSpec((1, _TQ, _NBR), lambda i, q: (i, q, 0)),
        out_shape=jax.ShapeDtypeStruct((b, n, _NBR), jnp.int32),
    )(xt, xf)


def _make_gather(tot, cp):
    info = plsc.get_sparse_core_info()
    nw = info.num_cores * info.num_subcores
    per_w = tot // nw
    nchunk = per_w // 128
    mesh = plsc.VectorSubcoreMesh(core_axis_name="c", subcore_axis_name="s")

    @functools.partial(
        pl.kernel, mesh=mesh,
        out_type=jax.ShapeDtypeStruct((tot, cp), jnp.float32),
        scratch_types=[
            pltpu.VMEM((nchunk, 128), jnp.int32),
            pltpu.VMEM((512, cp), jnp.float32),
            pltpu.SemaphoreType.DMA,
        ],
    )
    def gath(table_hbm, idx_hbm, out_hbm, idx_v, rows_v, sem):
        wid = lax.axis_index("s") * info.num_cores + lax.axis_index("c")
        pltpu.sync_copy(idx_hbm.at[pl.ds(wid * nchunk, nchunk)], idx_v)
        for rnd in range(nchunk // 4):
            descs = []
            for q in range(4):
                cid = rnd * 4 + q
                descs.append(pltpu.async_copy(
                    table_hbm.at[idx_v.at[cid]],
                    rows_v.at[pl.ds(q * 128, 128)], sem))
            for dsc in descs:
                dsc.wait()
            off = wid * per_w + rnd * 512
            pltpu.sync_copy(rows_v, out_hbm.at[pl.ds(off, 512)])

    return gath


_CP = 128


def _conv_body(xt_ref, g_ref, w0_ref, wr_ref, bias_ref, out_ref):
    acc = lax.dot_general(xt_ref[0], w0_ref[...], (((1,), (0,)), ((), ())),
                          preferred_element_type=jnp.float32)
    acc += lax.dot_general(g_ref[0], wr_ref[...], (((1,), (0,)), ((), ())),
                           preferred_element_type=jnp.float32)
    acc += bias_ref[...]
    out_ref[0] = acc.T


def _conv_call(xt, g2, w0, wr, bias):
    b, n, c = xt.shape
    return pl.pallas_call(
        _conv_body,
        grid=(b, n // _TR),
        in_specs=[
            pl.BlockSpec((1, _TR, c), lambda i, t: (i, t, 0)),
            pl.BlockSpec((1, _TR, _NBR * _CP), lambda i, t: (i, t, 0)),
            pl.BlockSpec((c, c), lambda i, t: (0, 0)),
            pl.BlockSpec((_NBR * _CP, c), lambda i, t: (0, 0)),
            pl.BlockSpec((1, c), lambda i, t: (0, 0)),
        ],
        out_specs=pl.BlockSpec((1, c, _TR), lambda i, t: (i, 0, t)),
        out_shape=jax.ShapeDtypeStruct((b, c, n), jnp.float32),
    )(xt, g2, w0, wr, bias)


def kernel(x, conv_w, conv_b):
    b, c, h, w = x.shape
    n = h * w
    xf = x.reshape(b, c, n)
    xt = jnp.transpose(xf, (0, 2, 1))

    idx = _topk_call(xt, xf)

    table = jnp.pad(xt.reshape(b * n, c), ((0, 0), (0, _CP - c)))
    flat_idx = idx.reshape(b * n * _NBR // 128, 128)
    g = _make_gather(b * n * _NBR, _CP)(table, flat_idx)
    g2 = g.reshape(b, n, _NBR * _CP)

    wt = jnp.transpose(conv_w, (2, 1, 0))
    w0 = wt[0]
    wr = jnp.pad(wt[1:], ((0, 0), (0, _CP - c), (0, 0))).reshape(
        _NBR * _CP, conv_w.shape[0])

    out = _conv_call(xt, g2, w0, wr, conv_b.reshape(1, conv_w.shape[0]))
    return out.reshape(b, conv_w.shape[0], h, w)


# speedup vs baseline: 242.6090x; 1.1640x over previous
import functools

import jax
import jax.numpy as jnp
from jax import lax
from jax.experimental import pallas as pl
from jax.experimental.pallas import tpu as pltpu
from jax.experimental.pallas import tpu_sc as plsc

_K = 9
_NBR = _K - 1
_TQ = 512
_TR = 512
_CP = 128


def _topk_body(qt_ref, xf_ref, idx_ref, tab_ref):
    b = pl.program_id(0)
    qi = pl.program_id(1)
    qt = qt_ref[0]
    xf = xf_ref[0]
    c, n = xf.shape
    dot = lax.dot_general(qt, xf, (((0,), (0,)), ((), ())),
                          preferred_element_type=jnp.float32)
    nk = jnp.sum(xf * xf, axis=0, keepdims=True)
    r = nk - 2.0 * dot
    lane = lax.broadcasted_iota(jnp.int32, r.shape, 1)
    row = qi * _TQ + lax.broadcasted_iota(jnp.int32, r.shape, 0)
    inf = jnp.float32(jnp.inf)
    r = jnp.where(lane == row, inf, r)
    lane_f = lane.astype(jnp.float32)
    lane8 = lax.broadcasted_iota(jnp.int32, (r.shape[0], _NBR), 1)
    acc = jnp.zeros((r.shape[0], _NBR), jnp.float32)
    for k in range(_NBR):
        m = jnp.min(r, axis=1, keepdims=True)
        i_f = jnp.min(jnp.where(r == m, lane_f, inf), axis=1,
                      keepdims=True)
        acc = jnp.where(lane8 == k, i_f, acc)
        r = jnp.where(lane_f == i_f, inf, r)
    idx_ref[0] = acc.astype(jnp.int32) + b * n
    xt = qt.T
    tab_ref[0] = jnp.concatenate(
        [xt, jnp.zeros((xt.shape[0], _CP - c), jnp.float32)], axis=1)


def _topk_call(xf):
    b, c, n = xf.shape
    return pl.pallas_call(
        _topk_body,
        grid=(b, n // _TQ),
        in_specs=[
            pl.BlockSpec((1, c, _TQ), lambda i, q: (i, 0, q)),
            pl.BlockSpec((1, c, n), lambda i, q: (i, 0, 0)),
        ],
        out_specs=[
            pl.BlockSpec((1, _TQ, _NBR), lambda i, q: (i, q, 0)),
            pl.BlockSpec((1, _TQ, _CP), lambda i, q: (i, q, 0)),
        ],
        out_shape=[
            jax.ShapeDtypeStruct((b, n, _NBR), jnp.int32),
            jax.ShapeDtypeStruct((b, n, _CP), jnp.float32),
        ],
    )(xf, xf)


def _make_gather(tot, cp):
    info = plsc.get_sparse_core_info()
    nw = info.num_cores * info.num_subcores
    per_w = tot // nw
    nchunk = per_w // 128
    nbuf = 4
    mesh = plsc.VectorSubcoreMesh(core_axis_name="c", subcore_axis_name="s")

    @functools.partial(
        pl.kernel, mesh=mesh,
        out_type=jax.ShapeDtypeStruct((tot, cp), jnp.float32),
        scratch_types=[
            pltpu.VMEM((nchunk, 128), jnp.int32),
            pltpu.VMEM((nbuf, 128, cp), jnp.float32),
            pltpu.SemaphoreType.DMA,
            pltpu.SemaphoreType.DMA,
        ],
    )
    def gath(table_hbm, idx_hbm, out_hbm, idx_v, rows_v, gsem, wsem):
        wid = lax.axis_index("s") * info.num_cores + lax.axis_index("c")
        base = wid * per_w
        pltpu.sync_copy(idx_hbm.at[pl.ds(wid * nchunk, nchunk)], idx_v)
        gd = [None] * nchunk
        wd = [None] * nchunk
        gd[0] = pltpu.async_copy(table_hbm.at[idx_v.at[0]], rows_v.at[0], gsem)
        for ck in range(nchunk):
            if ck + 1 < nchunk:
                bn = (ck + 1) % nbuf
                if ck + 1 >= nbuf:
                    wd[ck + 1 - nbuf].wait()
                gd[ck + 1] = pltpu.async_copy(
                    table_hbm.at[idx_v.at[ck + 1]], rows_v.at[bn], gsem)
            gd[ck].wait()
            wd[ck] = pltpu.async_copy(
                rows_v.at[ck % nbuf],
                out_hbm.at[pl.ds(base + ck * 128, 128)], wsem)
        for ck in range(nchunk - nbuf, nchunk):
            wd[ck].wait()

    return gath


def _conv_body(xfq_ref, g_ref, w0_ref, wr_ref, bias_ref, out_ref):
    acc = lax.dot_general(xfq_ref[0], w0_ref[...], (((0,), (0,)), ((), ())),
                          preferred_element_type=jnp.float32)
    acc += lax.dot_general(g_ref[0], wr_ref[...], (((1,), (0,)), ((), ())),
                           preferred_element_type=jnp.float32)
    acc += bias_ref[...]
    out_ref[0] = acc.T


def _conv_call(xf, g2, w0, wr, bias):
    b, c, n = xf.shape
    co = w0.shape[1]
    return pl.pallas_call(
        _conv_body,
        grid=(b, n // _TR),
        in_specs=[
            pl.BlockSpec((1, c, _TR), lambda i, t: (i, 0, t)),
            pl.BlockSpec((1, _TR, _NBR * _CP), lambda i, t: (i, t, 0)),
            pl.BlockSpec((c, co), lambda i, t: (0, 0)),
            pl.BlockSpec((_NBR * _CP, co), lambda i, t: (0, 0)),
            pl.BlockSpec((1, co), lambda i, t: (0, 0)),
        ],
        out_specs=pl.BlockSpec((1, co, _TR), lambda i, t: (i, 0, t)),
        out_shape=jax.ShapeDtypeStruct((b, co, n), jnp.float32),
    )(xf, g2, w0, wr, bias)


def kernel(x, conv_w, conv_b):
    b, c, h, w = x.shape
    n = h * w
    co = conv_w.shape[0]
    xf = x.reshape(b, c, n)

    idx, table = _topk_call(xf)

    flat_idx = idx.reshape(b * n * _NBR // 128, 128)
    g = _make_gather(b * n * _NBR, _CP)(table.reshape(b * n, _CP), flat_idx)
    g2 = g.reshape(b, n, _NBR * _CP)

    wt = jnp.transpose(conv_w, (2, 1, 0))
    w0 = wt[0]
    wr = jnp.pad(wt[1:], ((0, 0), (0, _CP - c), (0, 0))).reshape(_NBR * _CP, co)

    out = _conv_call(xf, g2, w0, wr, conv_b.reshape(1, co))
    return out.reshape(b, co, h, w)


# speedup vs baseline: 253.2517x; 1.0439x over previous
import functools

import jax
import jax.numpy as jnp
from jax import lax
from jax.experimental import pallas as pl
from jax.experimental.pallas import tpu as pltpu
from jax.experimental.pallas import tpu_sc as plsc

_K = 9
_NBR = _K - 1
_TQ = 512
_TR = 512
_CP = 128


def _topk_body(qt_ref, xf_ref, idx_ref, tab_ref):
    qi = pl.program_id(0)
    qt = qt_ref[0]
    xf = xf_ref[0]
    c, n = xf.shape
    dot = lax.dot_general(qt, xf, (((0,), (0,)), ((), ())),
                          preferred_element_type=jnp.float32)
    nk = jnp.sum(xf * xf, axis=0, keepdims=True)
    r = nk - 2.0 * dot
    lane = lax.broadcasted_iota(jnp.int32, r.shape, 1)
    row = qi * _TQ + lax.broadcasted_iota(jnp.int32, r.shape, 0)
    inf = jnp.float32(jnp.inf)
    r = jnp.where(lane == row, inf, r)
    lane_f = lane.astype(jnp.float32)
    lane8 = lax.broadcasted_iota(jnp.int32, (r.shape[0], _NBR), 1)
    acc = jnp.zeros((r.shape[0], _NBR), jnp.float32)
    for k in range(_NBR):
        m = jnp.min(r, axis=1, keepdims=True)
        i_f = jnp.min(jnp.where(r == m, lane_f, inf), axis=1,
                      keepdims=True)
        acc = jnp.where(lane8 == k, i_f, acc)
        r = jnp.where(lane_f == i_f, inf, r)
    idx_ref[...] = acc.astype(jnp.int32)
    xt = qt.T
    tab_ref[...] = jnp.concatenate(
        [xt, jnp.zeros((xt.shape[0], _CP - c), jnp.float32)], axis=1)


def _topk_call(xf, b):
    _, c, n = xf.shape
    return pl.pallas_call(
        _topk_body,
        grid=(n // _TQ,),
        in_specs=[
            pl.BlockSpec((1, c, _TQ), lambda q: (b, 0, q)),
            pl.BlockSpec((1, c, n), lambda q: (b, 0, 0)),
        ],
        out_specs=[
            pl.BlockSpec((_TQ, _NBR), lambda q: (q, 0)),
            pl.BlockSpec((_TQ, _CP), lambda q: (q, 0)),
        ],
        out_shape=[
            jax.ShapeDtypeStruct((n, _NBR), jnp.int32),
            jax.ShapeDtypeStruct((n, _CP), jnp.float32),
        ],
    )(xf, xf)


def _make_gather(tot, cp):
    info = plsc.get_sparse_core_info()
    nw = info.num_cores * info.num_subcores
    per_w = tot // nw
    nchunk = per_w // 128
    nbuf = min(6, nchunk)
    ahead = nbuf - 2
    mesh = plsc.VectorSubcoreMesh(core_axis_name="c", subcore_axis_name="s")

    @functools.partial(
        pl.kernel, mesh=mesh,
        out_type=jax.ShapeDtypeStruct((tot, cp), jnp.float32),
        scratch_types=[
            pltpu.VMEM((nchunk, 128), jnp.int32),
            pltpu.VMEM((nbuf, 128, cp), jnp.float32),
            pltpu.SemaphoreType.DMA,
            pltpu.SemaphoreType.DMA,
        ],
    )
    def gath(table_hbm, idx_hbm, out_hbm, idx_v, rows_v, gsem, wsem):
        wid = lax.axis_index("s") * info.num_cores + lax.axis_index("c")
        base = wid * per_w
        pltpu.sync_copy(idx_hbm.at[pl.ds(wid * nchunk, nchunk)], idx_v)
        gd = [None] * nchunk
        wd = [None] * nchunk
        for ck in range(min(ahead, nchunk)):
            gd[ck] = pltpu.async_copy(
                table_hbm.at[idx_v.at[ck]], rows_v.at[ck % nbuf], gsem)
        for ck in range(nchunk):
            nx = ck + ahead
            if nx < nchunk:
                if nx >= nbuf:
                    wd[nx - nbuf].wait()
                gd[nx] = pltpu.async_copy(
                    table_hbm.at[idx_v.at[nx]], rows_v.at[nx % nbuf], gsem)
            gd[ck].wait()
            wd[ck] = pltpu.async_copy(
                rows_v.at[ck % nbuf],
                out_hbm.at[pl.ds(base + ck * 128, 128)], wsem)
        for ck in range(max(0, nchunk - nbuf), nchunk):
            wd[ck].wait()

    return gath


def _conv_body(xfq_ref, g_ref, w0_ref, wr_ref, bias_ref, out_ref):
    acc = lax.dot_general(xfq_ref[0], w0_ref[...], (((0,), (0,)), ((), ())),
                          preferred_element_type=jnp.float32)
    acc += lax.dot_general(g_ref[...], wr_ref[...], (((1,), (0,)), ((), ())),
                           preferred_element_type=jnp.float32)
    acc += bias_ref[...]
    out_ref[...] = acc.T


def _conv_call(xf, g2, w0, wr, bias, b):
    _, c, n = xf.shape
    co = w0.shape[1]
    return pl.pallas_call(
        _conv_body,
        grid=(n // _TR,),
        in_specs=[
            pl.BlockSpec((1, c, _TR), lambda t: (b, 0, t)),
            pl.BlockSpec((_TR, _NBR * _CP), lambda t: (t, 0)),
            pl.BlockSpec((c, co), lambda t: (0, 0)),
            pl.BlockSpec((_NBR * _CP, co), lambda t: (0, 0)),
            pl.BlockSpec((1, co), lambda t: (0, 0)),
        ],
        out_specs=pl.BlockSpec((co, _TR), lambda t: (0, t)),
        out_shape=jax.ShapeDtypeStruct((co, n), jnp.float32),
    )(xf, g2, w0, wr, bias)


def kernel(x, conv_w, conv_b):
    b, c, h, w = x.shape
    n = h * w
    co = conv_w.shape[0]
    xf = x.reshape(b, c, n)

    wt = jnp.transpose(conv_w, (2, 1, 0))
    w0 = wt[0]
    wr = jnp.pad(wt[1:], ((0, 0), (0, _CP - c), (0, 0))).reshape(_NBR * _CP, co)
    bias = conv_b.reshape(1, co)
    gath = _make_gather(n * _NBR, _CP)

    outs = []
    for bi in range(b):
        idx, table = _topk_call(xf, bi)
        flat_idx = idx.reshape(n * _NBR // 128, 128)
        g = gath(table, flat_idx)
        g2 = g.reshape(n, _NBR * _CP)
        outs.append(_conv_call(xf, g2, w0, wr, bias, bi))
    return jnp.stack(outs).reshape(b, co, h, w)
